# Initial kernel scaffold; baseline (speedup 1.0000x reference)
import jax
import jax.numpy as jnp
from jax.experimental import pallas as pl


def kernel(x, edge_index, W1, b1, W2, b2, Wp1, bp1, Wp2, bp2):
    raise NotImplementedError("write your pallas kernel here")


# speedup vs baseline: 4.9147x; 4.9147x over previous
import jax
import jax.numpy as jnp
from jax import lax
from jax.experimental import pallas as pl
from jax.experimental.pallas import tpu as pltpu
from jax.experimental.pallas import tpu_sc as plsc

N = 10000
E = 320000
D = 128
H = 128
PH = 256
PO = 128

NC = 2
NS = 16
DH = D // NC
EW = E // NS
C = 80
NCHUNK = EW // C
NP = 10240
RPS = NP // NS
ZR = 128


def _sc_pass(with_deg):
    mesh = plsc.VectorSubcoreMesh(core_axis_name="c", subcore_axis_name="s")
    out_type = [jax.ShapeDtypeStruct((NC, NP, DH), jnp.float32)]
    if with_deg:
        out_type.append(jax.ShapeDtypeStruct((NP, 16), jnp.float32))
    scratch = [
        pltpu.VMEM((EW,), jnp.int32),
        pltpu.VMEM((NCHUNK, C), jnp.int32),
        pltpu.VMEM((C, DH), jnp.float32),
        pltpu.VMEM((ZR, DH), jnp.float32),
        pltpu.VMEM((C, 16), jnp.float32),
        pltpu.VMEM((ZR, 16), jnp.float32),
        pltpu.VMEM_SHARED((NP, DH), jnp.float32),
        pltpu.VMEM_SHARED((NP, 16), jnp.float32),
        pltpu.SemaphoreType.DMA,
    ]

    def body(xh_hbm, src_hbm, dst_hbm, *rest):
        if with_deg:
            out_acc, out_deg = rest[0], rest[1]
            scr = rest[2:]
        else:
            out_acc = rest[0]
            scr = rest[1:]
        src_v, dst_v, rows_v, zbuf, ones_v, zdeg, acc_sh, deg_sh, sem = scr

        cid = lax.axis_index("c")
        sid = lax.axis_index("s")

        z16 = jnp.zeros((16,), jnp.float32)
        ones16 = jnp.full((16,), 1.0, jnp.float32)

        def zero_zbuf(i, carry):
            for j in range(DH // 16):
                zbuf[i, pl.ds(j * 16, 16)] = z16
            return carry

        lax.fori_loop(0, ZR, zero_zbuf, 0)
        if with_deg:
            def fill_ones(i, carry):
                ones_v[i, pl.ds(0, 16)] = ones16
                return carry

            lax.fori_loop(0, C, fill_ones, 0)

            def zero_zdeg(i, carry):
                zdeg[i, pl.ds(0, 16)] = z16
                return carry

            lax.fori_loop(0, ZR, zero_zdeg, 0)
        for k in range(RPS // ZR):
            pltpu.sync_copy(zbuf, acc_sh.at[pl.ds(sid * RPS + k * ZR, ZR)])
            if with_deg:
                @pl.when(cid == 0)
                def _(k=k):
                    pltpu.sync_copy(zdeg,
                                    deg_sh.at[pl.ds(sid * RPS + k * ZR, ZR)])

        pltpu.sync_copy(src_hbm.at[pl.ds(sid * EW, EW)], src_v)
        pltpu.sync_copy(dst_hbm.at[sid], dst_v)
        plsc.subcore_barrier()

        def step(g, carry):
            pltpu.async_copy(
                xh_hbm.at[cid].at[src_v.at[pl.ds(g * C, C)]], rows_v,
                sem).wait()
            pltpu.sync_copy(rows_v, acc_sh.at[dst_v.at[g]], add=True)
            if with_deg:
                @pl.when(cid == 0)
                def _():
                    pltpu.sync_copy(ones_v, deg_sh.at[dst_v.at[g]], add=True)
            return carry

        lax.fori_loop(0, NCHUNK, step, 0)
        plsc.subcore_barrier()

        pltpu.sync_copy(acc_sh.at[pl.ds(sid * RPS, RPS)],
                        out_acc.at[cid, pl.ds(sid * RPS, RPS)])
        if with_deg:
            @pl.when(cid == 0)
            def _():
                pltpu.sync_copy(deg_sh.at[pl.ds(sid * RPS, RPS)],
                                out_deg.at[pl.ds(sid * RPS, RPS)])

    return pl.kernel(body, out_type=tuple(out_type), mesh=mesh,
                     scratch_types=scratch,
                     compiler_params=pltpu.CompilerParams(
                         use_tc_tiling_on_sc=False))


_sc_pass1 = _sc_pass(True)
_sc_pass2 = _sc_pass(False)

TN = 1000


def _tc1_body(p_ref, dg_ref, w_ref, b_ref, h_ref):
    inv = 1.0 / jnp.maximum(dg_ref[:, 0], 1.0)
    agg = jnp.concatenate([p_ref[0], p_ref[1]], axis=1) * inv[:, None]
    h = jnp.dot(agg, w_ref[...], preferred_element_type=jnp.float32,
                precision=lax.Precision.HIGHEST)
    h = jnp.maximum(h + b_ref[...], 0.0)
    h_ref[0] = h[:, :DH]
    h_ref[1] = h[:, DH:]


def _tc1(partials, degp, W1, b1):
    return pl.pallas_call(
        _tc1_body,
        grid=(N // TN,),
        in_specs=[
            pl.BlockSpec((NC, TN, DH), lambda i: (0, i, 0)),
            pl.BlockSpec((TN, 16), lambda i: (i, 0)),
            pl.BlockSpec((D, H), lambda i: (0, 0)),
            pl.BlockSpec((1, H), lambda i: (0, 0)),
        ],
        out_specs=pl.BlockSpec((NC, TN, DH), lambda i: (0, i, 0)),
        out_shape=jax.ShapeDtypeStruct((NC, N, DH), jnp.float32),
    )(partials, degp, W1, b1.reshape(1, H))


def _tc2_body(p_ref, dg_ref, w2_ref, b2_ref, wp1_ref, bp1_ref, wp2_ref,
              bp2_ref, loc_ref, g_ref):
    i = pl.program_id(0)
    inv = 1.0 / jnp.maximum(dg_ref[:, 0], 1.0)
    agg = jnp.concatenate([p_ref[0], p_ref[1]], axis=1) * inv[:, None]
    h2 = jnp.maximum(
        jnp.dot(agg, w2_ref[...], preferred_element_type=jnp.float32,
                precision=lax.Precision.HIGHEST) + b2_ref[...], 0.0)
    t = jnp.maximum(
        jnp.dot(h2, wp1_ref[...], preferred_element_type=jnp.float32,
                precision=lax.Precision.HIGHEST) + bp1_ref[...], 0.0)
    loc = jnp.dot(t, wp2_ref[...], preferred_element_type=jnp.float32,
                  precision=lax.Precision.HIGHEST) + bp2_ref[...]
    loc_ref[...] = loc

    @pl.when(i == 0)
    def _():
        g_ref[...] = jnp.zeros_like(g_ref)

    g_ref[...] += jnp.sum(loc, axis=0, keepdims=True) * (1.0 / N)


def _tc2(partials, degp, W2, b2, Wp1, bp1, Wp2, bp2):
    return pl.pallas_call(
        _tc2_body,
        grid=(N // TN,),
        in_specs=[
            pl.BlockSpec((NC, TN, DH), lambda i: (0, i, 0)),
            pl.BlockSpec((TN, 16), lambda i: (i, 0)),
            pl.BlockSpec((H, H), lambda i: (0, 0)),
            pl.BlockSpec((1, H), lambda i: (0, 0)),
            pl.BlockSpec((H, PH), lambda i: (0, 0)),
            pl.BlockSpec((1, PH), lambda i: (0, 0)),
            pl.BlockSpec((PH, PO), lambda i: (0, 0)),
            pl.BlockSpec((1, PO), lambda i: (0, 0)),
        ],
        out_specs=[
            pl.BlockSpec((TN, PO), lambda i: (i, 0)),
            pl.BlockSpec((1, PO), lambda i: (0, 0)),
        ],
        out_shape=[
            jax.ShapeDtypeStruct((N, PO), jnp.float32),
            jax.ShapeDtypeStruct((1, PO), jnp.float32),
        ],
    )(partials, degp, W2, b2.reshape(1, H), Wp1, bp1.reshape(1, PH),
      Wp2, bp2.reshape(1, PO))


def kernel(x, edge_index, W1, b1, W2, b2, Wp1, bp1, Wp2, bp2):
    src = edge_index[0]
    dst3 = edge_index[1].reshape(NS, NCHUNK, C)
    xh = jnp.stack([x[:, :DH], x[:, DH:]])
    p1, degp = _sc_pass1(xh, src, dst3)
    h1 = _tc1(p1, degp, W1, b1)
    (p2,) = _sc_pass2(h1, src, dst3)
    loc, ge = _tc2(p2, degp, W2, b2, Wp1, bp1, Wp2, bp2)
    return (ge, loc)


# speedup vs baseline: 8.5858x; 1.7470x over previous
import jax
import jax.numpy as jnp
from jax import lax
from jax.experimental import pallas as pl
from jax.experimental.pallas import tpu as pltpu
from jax.experimental.pallas import tpu_sc as plsc

N = 10000
E = 320000
D = 128
H = 128
PH = 256
PO = 128

NC = 2
NS = 16
DH = D // NC
EW = E // NS
C = 80
NCHUNK = EW // C
NP = 10240
RPS = NP // NS
ZR = 128
B = 5
NG = NCHUNK // B


def _sc_pass(with_deg):
    mesh = plsc.VectorSubcoreMesh(core_axis_name="c", subcore_axis_name="s")
    out_type = [jax.ShapeDtypeStruct((NC, NP, DH), jnp.float32)]
    if with_deg:
        out_type.append(jax.ShapeDtypeStruct((NP, 16), jnp.float32))
    scratch = [
        pltpu.VMEM((EW,), jnp.int32),
        pltpu.VMEM((NCHUNK, C), jnp.int32),
        pltpu.VMEM((B, C, DH), jnp.float32),
        pltpu.VMEM((ZR, DH), jnp.float32),
        pltpu.VMEM((C, 16), jnp.float32),
        pltpu.VMEM((ZR, 16), jnp.float32),
        pltpu.VMEM_SHARED((NP, DH), jnp.float32),
        pltpu.VMEM_SHARED((NP, 16), jnp.float32),
        pltpu.SemaphoreType.DMA,
    ]

    def body(xh_hbm, src_hbm, dst_hbm, *rest):
        if with_deg:
            out_acc, out_deg = rest[0], rest[1]
            scr = rest[2:]
        else:
            out_acc = rest[0]
            scr = rest[1:]
        (src_v, dst_v, rows_v, zbuf, ones_v, zdeg, acc_sh, deg_sh,
         gsem) = scr

        cid = lax.axis_index("c")
        sid = lax.axis_index("s")

        z16 = jnp.zeros((16,), jnp.float32)
        ones16 = jnp.full((16,), 1.0, jnp.float32)

        def zero_zbuf(i, carry):
            for j in range(DH // 16):
                zbuf[i, pl.ds(j * 16, 16)] = z16
            return carry

        lax.fori_loop(0, ZR, zero_zbuf, 0)
        if with_deg:
            def fill_ones(i, carry):
                ones_v[i, pl.ds(0, 16)] = ones16
                return carry

            lax.fori_loop(0, C, fill_ones, 0)

            def zero_zdeg(i, carry):
                zdeg[i, pl.ds(0, 16)] = z16
                return carry

            lax.fori_loop(0, ZR, zero_zdeg, 0)
        for k in range(RPS // ZR):
            pltpu.sync_copy(zbuf, acc_sh.at[pl.ds(sid * RPS + k * ZR, ZR)])
            if with_deg:
                pltpu.sync_copy(zdeg,
                                deg_sh.at[pl.ds(sid * RPS + k * ZR, ZR)])

        pltpu.sync_copy(src_hbm.at[pl.ds(sid * EW, EW)], src_v)
        pltpu.sync_copy(dst_hbm.at[sid], dst_v)
        plsc.subcore_barrier()

        def step(g, carry):
            descs = []
            for b in range(B):
                descs.append(pltpu.async_copy(
                    xh_hbm.at[cid].at[src_v.at[pl.ds((g * B + b) * C, C)]],
                    rows_v.at[b], gsem))
            for b in range(B):
                descs[b].wait()
                pltpu.sync_copy(rows_v.at[b],
                                acc_sh.at[dst_v.at[g * B + b]], add=True)
                if with_deg:
                    @pl.when(cid == 0)
                    def _():
                        pltpu.sync_copy(ones_v,
                                        deg_sh.at[dst_v.at[g * B + b]],
                                        add=True)
            return carry

        lax.fori_loop(0, NG, step, 0)
        plsc.subcore_barrier()

        pltpu.sync_copy(acc_sh.at[pl.ds(sid * RPS, RPS)],
                        out_acc.at[cid, pl.ds(sid * RPS, RPS)])
        if with_deg:
            @pl.when(cid == 0)
            def _():
                pltpu.sync_copy(deg_sh.at[pl.ds(sid * RPS, RPS)],
                                out_deg.at[pl.ds(sid * RPS, RPS)])

    return pl.kernel(body, out_type=tuple(out_type), mesh=mesh,
                     scratch_types=scratch,
                     compiler_params=pltpu.CompilerParams(
                         use_tc_tiling_on_sc=False))


_sc_pass1 = _sc_pass(True)
_sc_pass2 = _sc_pass(False)

TN = 1000


def _tc1_body(p_ref, dg_ref, w_ref, b_ref, h_ref):
    inv = 1.0 / jnp.maximum(dg_ref[:, 0], 1.0)
    agg = jnp.concatenate([p_ref[0], p_ref[1]], axis=1) * inv[:, None]
    h = jnp.dot(agg, w_ref[...], preferred_element_type=jnp.float32,
                precision=lax.Precision.HIGHEST)
    h = jnp.maximum(h + b_ref[...], 0.0)
    h_ref[0] = h[:, :DH]
    h_ref[1] = h[:, DH:]


def _tc1(partials, degp, W1, b1):
    return pl.pallas_call(
        _tc1_body,
        grid=(N // TN,),
        in_specs=[
            pl.BlockSpec((NC, TN, DH), lambda i: (0, i, 0)),
            pl.BlockSpec((TN, 16), lambda i: (i, 0)),
            pl.BlockSpec((D, H), lambda i: (0, 0)),
            pl.BlockSpec((1, H), lambda i: (0, 0)),
        ],
        out_specs=pl.BlockSpec((NC, TN, DH), lambda i: (0, i, 0)),
        out_shape=jax.ShapeDtypeStruct((NC, N, DH), jnp.float32),
    )(partials, degp, W1, b1.reshape(1, H))


def _tc2_body(p_ref, dg_ref, w2_ref, b2_ref, wp1_ref, bp1_ref, wp2_ref,
              bp2_ref, loc_ref, g_ref):
    i = pl.program_id(0)
    inv = 1.0 / jnp.maximum(dg_ref[:, 0], 1.0)
    agg = jnp.concatenate([p_ref[0], p_ref[1]], axis=1) * inv[:, None]
    h2 = jnp.maximum(
        jnp.dot(agg, w2_ref[...], preferred_element_type=jnp.float32,
                precision=lax.Precision.HIGHEST) + b2_ref[...], 0.0)
    t = jnp.maximum(
        jnp.dot(h2, wp1_ref[...], preferred_element_type=jnp.float32,
                precision=lax.Precision.HIGHEST) + bp1_ref[...], 0.0)
    loc = jnp.dot(t, wp2_ref[...], preferred_element_type=jnp.float32,
                  precision=lax.Precision.HIGHEST) + bp2_ref[...]
    loc_ref[...] = loc

    @pl.when(i == 0)
    def _():
        g_ref[...] = jnp.zeros_like(g_ref)

    g_ref[...] += jnp.sum(loc, axis=0, keepdims=True) * (1.0 / N)


def _tc2(partials, degp, W2, b2, Wp1, bp1, Wp2, bp2):
    return pl.pallas_call(
        _tc2_body,
        grid=(N // TN,),
        in_specs=[
            pl.BlockSpec((NC, TN, DH), lambda i: (0, i, 0)),
            pl.BlockSpec((TN, 16), lambda i: (i, 0)),
            pl.BlockSpec((H, H), lambda i: (0, 0)),
            pl.BlockSpec((1, H), lambda i: (0, 0)),
            pl.BlockSpec((H, PH), lambda i: (0, 0)),
            pl.BlockSpec((1, PH), lambda i: (0, 0)),
            pl.BlockSpec((PH, PO), lambda i: (0, 0)),
            pl.BlockSpec((1, PO), lambda i: (0, 0)),
        ],
        out_specs=[
            pl.BlockSpec((TN, PO), lambda i: (i, 0)),
            pl.BlockSpec((1, PO), lambda i: (0, 0)),
        ],
        out_shape=[
            jax.ShapeDtypeStruct((N, PO), jnp.float32),
            jax.ShapeDtypeStruct((1, PO), jnp.float32),
        ],
    )(partials, degp, W2, b2.reshape(1, H), Wp1, bp1.reshape(1, PH),
      Wp2, bp2.reshape(1, PO))


def kernel(x, edge_index, W1, b1, W2, b2, Wp1, bp1, Wp2, bp2):
    src = edge_index[0]
    dst3 = edge_index[1].reshape(NS, NCHUNK, C)
    xh = jnp.stack([x[:, :DH], x[:, DH:]])
    p1, degp = _sc_pass1(xh, src, dst3)
    h1 = _tc1(p1, degp, W1, b1)
    (p2,) = _sc_pass2(h1, src, dst3)
    loc, ge = _tc2(p2, degp, W2, b2, Wp1, bp1, Wp2, bp2)
    return (ge, loc)


# speedup vs baseline: 8.8251x; 1.0279x over previous
import jax
import jax.numpy as jnp
from jax import lax
from jax.experimental import pallas as pl
from jax.experimental.pallas import tpu as pltpu
from jax.experimental.pallas import tpu_sc as plsc

N = 10000
E = 320000
D = 128
H = 128
PH = 256
PO = 128

NC = 2
NS = 16
DH = D // NC
EW = E // NS
C = 80
NCHUNK = EW // C
NP = 10240
RPS = NP // NS
ZR = 128
B = 5
NG = NCHUNK // B


def _sc_pass(with_deg):
    mesh = plsc.VectorSubcoreMesh(core_axis_name="c", subcore_axis_name="s")
    out_type = [jax.ShapeDtypeStruct((NC, NP, DH), jnp.float32)]
    if with_deg:
        out_type.append(jax.ShapeDtypeStruct((NC, NP, 16), jnp.float32))
    scratch = [
        pltpu.VMEM((EW,), jnp.int32),
        pltpu.VMEM((NCHUNK, C), jnp.int32),
        pltpu.VMEM((B, C, DH), jnp.float32),
        pltpu.VMEM((ZR, DH), jnp.float32),
        pltpu.VMEM((C, 16), jnp.float32),
        pltpu.VMEM((ZR, 16), jnp.float32),
        pltpu.VMEM_SHARED((NP, DH), jnp.float32),
        pltpu.VMEM_SHARED((NP, 16), jnp.float32),
        pltpu.SemaphoreType.DMA,
        pltpu.SemaphoreType.DMA,
        pltpu.SemaphoreType.DMA,
    ]

    def body(xh_hbm, src_hbm, dst_hbm, *rest):
        if with_deg:
            out_acc, out_deg = rest[0], rest[1]
            scr = rest[2:]
        else:
            out_acc = rest[0]
            scr = rest[1:]
        (src_v, dst_v, rows_v, zbuf, ones_v, zdeg, acc_sh, deg_sh,
         gsem, ssem, dsem) = scr

        cid = lax.axis_index("c")
        sid = lax.axis_index("s")

        z16 = jnp.zeros((16,), jnp.float32)
        ones16 = jnp.full((16,), 1.0, jnp.float32)

        def zero_zbuf(i, carry):
            for j in range(DH // 16):
                zbuf[i, pl.ds(j * 16, 16)] = z16
            return carry

        lax.fori_loop(0, ZR, zero_zbuf, 0)
        if with_deg:
            def fill_ones(i, carry):
                ones_v[i, pl.ds(0, 16)] = ones16
                return carry

            lax.fori_loop(0, C, fill_ones, 0)

            def zero_zdeg(i, carry):
                zdeg[i, pl.ds(0, 16)] = z16
                return carry

            lax.fori_loop(0, ZR, zero_zdeg, 0)
        for k in range(RPS // ZR):
            pltpu.sync_copy(zbuf, acc_sh.at[pl.ds(sid * RPS + k * ZR, ZR)])
            if with_deg:
                pltpu.sync_copy(zdeg,
                                deg_sh.at[pl.ds(sid * RPS + k * ZR, ZR)])

        pltpu.sync_copy(src_hbm.at[pl.ds(sid * EW, EW)], src_v)
        pltpu.sync_copy(dst_hbm.at[sid], dst_v)
        plsc.subcore_barrier()

        def step(g, carry):
            gds = []
            for b in range(B):
                gds.append(pltpu.async_copy(
                    xh_hbm.at[cid].at[src_v.at[pl.ds((g * B + b) * C, C)]],
                    rows_v.at[b], gsem))
            for b in range(B):
                gds[b].wait()
                if with_deg:
                    @pl.when(cid == g % 2)
                    def _(b=b):
                        pltpu.async_copy(
                            ones_v, deg_sh.at[dst_v.at[g * B + b]], dsem,
                            add=True)
                pltpu.sync_copy(rows_v.at[b],
                                acc_sh.at[dst_v.at[g * B + b]], add=True)
            if with_deg:
                @pl.when(cid == g % 2)
                def _():
                    for b in range(B):
                        pltpu.make_async_copy(
                            ones_v, deg_sh.at[dst_v.at[g * B + b]],
                            dsem).wait()
            return carry

        lax.fori_loop(0, NG, step, 0)
        plsc.subcore_barrier()

        pltpu.sync_copy(acc_sh.at[pl.ds(sid * RPS, RPS)],
                        out_acc.at[cid, pl.ds(sid * RPS, RPS)])
        if with_deg:
            pltpu.sync_copy(deg_sh.at[pl.ds(sid * RPS, RPS)],
                            out_deg.at[cid, pl.ds(sid * RPS, RPS)])

    return pl.kernel(body, out_type=tuple(out_type), mesh=mesh,
                     scratch_types=scratch,
                     compiler_params=pltpu.CompilerParams(
                         use_tc_tiling_on_sc=False))


_sc_pass1 = _sc_pass(True)
_sc_pass2 = _sc_pass(False)

TN = 1000


def _tc1_body(p_ref, dg_ref, w_ref, b_ref, h_ref):
    inv = 1.0 / jnp.maximum(dg_ref[0, :, 0] + dg_ref[1, :, 0], 1.0)
    agg = jnp.concatenate([p_ref[0], p_ref[1]], axis=1) * inv[:, None]
    h = jnp.dot(agg, w_ref[...], preferred_element_type=jnp.float32,
                precision=lax.Precision.HIGHEST)
    h = jnp.maximum(h + b_ref[...], 0.0)
    h_ref[0] = h[:, :DH]
    h_ref[1] = h[:, DH:]


def _tc1(partials, degp, W1, b1):
    return pl.pallas_call(
        _tc1_body,
        grid=(N // TN,),
        in_specs=[
            pl.BlockSpec((NC, TN, DH), lambda i: (0, i, 0)),
            pl.BlockSpec((NC, TN, 16), lambda i: (0, i, 0)),
            pl.BlockSpec((D, H), lambda i: (0, 0)),
            pl.BlockSpec((1, H), lambda i: (0, 0)),
        ],
        out_specs=pl.BlockSpec((NC, TN, DH), lambda i: (0, i, 0)),
        out_shape=jax.ShapeDtypeStruct((NC, N, DH), jnp.float32),
    )(partials, degp, W1, b1.reshape(1, H))


def _tc2_body(p_ref, dg_ref, w2_ref, b2_ref, wp1_ref, bp1_ref, wp2_ref,
              bp2_ref, loc_ref, g_ref):
    i = pl.program_id(0)
    inv = 1.0 / jnp.maximum(dg_ref[0, :, 0] + dg_ref[1, :, 0], 1.0)
    agg = jnp.concatenate([p_ref[0], p_ref[1]], axis=1) * inv[:, None]
    h2 = jnp.maximum(
        jnp.dot(agg, w2_ref[...], preferred_element_type=jnp.float32,
                precision=lax.Precision.HIGHEST) + b2_ref[...], 0.0)
    t = jnp.maximum(
        jnp.dot(h2, wp1_ref[...], preferred_element_type=jnp.float32,
                precision=lax.Precision.HIGHEST) + bp1_ref[...], 0.0)
    loc = jnp.dot(t, wp2_ref[...], preferred_element_type=jnp.float32,
                  precision=lax.Precision.HIGHEST) + bp2_ref[...]
    loc_ref[...] = loc

    @pl.when(i == 0)
    def _():
        g_ref[...] = jnp.zeros_like(g_ref)

    g_ref[...] += jnp.sum(loc, axis=0, keepdims=True) * (1.0 / N)


def _tc2(partials, degp, W2, b2, Wp1, bp1, Wp2, bp2):
    return pl.pallas_call(
        _tc2_body,
        grid=(N // TN,),
        in_specs=[
            pl.BlockSpec((NC, TN, DH), lambda i: (0, i, 0)),
            pl.BlockSpec((NC, TN, 16), lambda i: (0, i, 0)),
            pl.BlockSpec((H, H), lambda i: (0, 0)),
            pl.BlockSpec((1, H), lambda i: (0, 0)),
            pl.BlockSpec((H, PH), lambda i: (0, 0)),
            pl.BlockSpec((1, PH), lambda i: (0, 0)),
            pl.BlockSpec((PH, PO), lambda i: (0, 0)),
            pl.BlockSpec((1, PO), lambda i: (0, 0)),
        ],
        out_specs=[
            pl.BlockSpec((TN, PO), lambda i: (i, 0)),
            pl.BlockSpec((1, PO), lambda i: (0, 0)),
        ],
        out_shape=[
            jax.ShapeDtypeStruct((N, PO), jnp.float32),
            jax.ShapeDtypeStruct((1, PO), jnp.float32),
        ],
    )(partials, degp, W2, b2.reshape(1, H), Wp1, bp1.reshape(1, PH),
      Wp2, bp2.reshape(1, PO))


def kernel(x, edge_index, W1, b1, W2, b2, Wp1, bp1, Wp2, bp2):
    src = edge_index[0]
    dst3 = edge_index[1].reshape(NS, NCHUNK, C)
    xh = jnp.stack([x[:, :DH], x[:, DH:]])
    p1, degp = _sc_pass1(xh, src, dst3)
    h1 = _tc1(p1, degp, W1, b1)
    (p2,) = _sc_pass2(h1, src, dst3)
    loc, ge = _tc2(p2, degp, W2, b2, Wp1, bp1, Wp2, bp2)
    return (ge, loc)


# speedup vs baseline: 9.5786x; 1.0854x over previous
import jax
import jax.numpy as jnp
from jax import lax
from jax.experimental import pallas as pl
from jax.experimental.pallas import tpu as pltpu
from jax.experimental.pallas import tpu_sc as plsc

N = 10000
E = 320000
D = 128
H = 128
PH = 256
PO = 128

NC = 2
NS = 16
DH = D // NC
EW = E // NS
C = 80
NCHUNK = EW // C
NP = 10240
RPS = NP // NS
ZR = 128
B = 5
NG = NCHUNK // B


def _sc_pass(with_deg):
    mesh = plsc.VectorSubcoreMesh(core_axis_name="c", subcore_axis_name="s")
    out_type = [jax.ShapeDtypeStruct((NC, NP, DH), jnp.float32)]
    if with_deg:
        out_type.append(jax.ShapeDtypeStruct((NC, NP, 16), jnp.float32))
    scratch = [
        pltpu.VMEM((EW,), jnp.int32),
        pltpu.VMEM((NCHUNK, C), jnp.int32),
        pltpu.VMEM((B, C, DH), jnp.float32),
        pltpu.VMEM((ZR, DH), jnp.float32),
        pltpu.VMEM((C, 16), jnp.float32),
        pltpu.VMEM((ZR, 16), jnp.float32),
        pltpu.VMEM_SHARED((NP, DH), jnp.float32),
        pltpu.VMEM_SHARED((NP, 16), jnp.float32),
        pltpu.SemaphoreType.DMA,
        pltpu.SemaphoreType.DMA,
        pltpu.SemaphoreType.DMA,
    ]

    def body(xh_hbm, src_hbm, dst_hbm, *rest):
        if with_deg:
            out_acc, out_deg = rest[0], rest[1]
            scr = rest[2:]
        else:
            out_acc = rest[0]
            scr = rest[1:]
        (src_v, dst_v, rows_v, zbuf, ones_v, zdeg, acc_sh, deg_sh,
         gsem, ssem, dsem) = scr

        cid = lax.axis_index("c")
        sid = lax.axis_index("s")

        z16 = jnp.zeros((16,), jnp.float32)
        ones16 = jnp.full((16,), 1.0, jnp.float32)

        def zero_zbuf(i, carry):
            for j in range(DH // 16):
                zbuf[i, pl.ds(j * 16, 16)] = z16
            return carry

        lax.fori_loop(0, ZR, zero_zbuf, 0)
        if with_deg:
            def fill_ones(i, carry):
                ones_v[i, pl.ds(0, 16)] = ones16
                return carry

            lax.fori_loop(0, C, fill_ones, 0)

            def zero_zdeg(i, carry):
                zdeg[i, pl.ds(0, 16)] = z16
                return carry

            lax.fori_loop(0, ZR, zero_zdeg, 0)
        for k in range(RPS // ZR):
            pltpu.sync_copy(zbuf, acc_sh.at[pl.ds(sid * RPS + k * ZR, ZR)])
            if with_deg:
                pltpu.sync_copy(zdeg,
                                deg_sh.at[pl.ds(sid * RPS + k * ZR, ZR)])

        pltpu.sync_copy(src_hbm.at[pl.ds(sid * EW, EW)], src_v)
        pltpu.sync_copy(dst_hbm.at[sid], dst_v)
        plsc.subcore_barrier()

        def step(g, carry):
            gds = []
            for b in range(B):
                gds.append(pltpu.async_copy(
                    xh_hbm.at[cid].at[src_v.at[pl.ds((g * B + b) * C, C)]],
                    rows_v.at[b], gsem))
            for b in range(B):
                gds[b].wait()
                if with_deg:
                    @pl.when(cid == g % 2)
                    def _(b=b):
                        pltpu.async_copy(
                            ones_v, deg_sh.at[dst_v.at[g * B + b]], dsem,
                            add=True)
                pltpu.sync_copy(rows_v.at[b],
                                acc_sh.at[dst_v.at[g * B + b]], add=True)
            if with_deg:
                @pl.when(cid == g % 2)
                def _():
                    for b in range(B):
                        pltpu.make_async_copy(
                            ones_v, deg_sh.at[dst_v.at[g * B + b]],
                            dsem).wait()
            return carry

        lax.fori_loop(0, NG, step, 0)
        plsc.subcore_barrier()

        pltpu.sync_copy(acc_sh.at[pl.ds(sid * RPS, RPS)],
                        out_acc.at[cid, pl.ds(sid * RPS, RPS)])
        if with_deg:
            pltpu.sync_copy(deg_sh.at[pl.ds(sid * RPS, RPS)],
                            out_deg.at[cid, pl.ds(sid * RPS, RPS)])

    return pl.kernel(body, out_type=tuple(out_type), mesh=mesh,
                     scratch_types=scratch,
                     compiler_params=pltpu.CompilerParams(
                         use_tc_tiling_on_sc=False))


_sc_pass1 = _sc_pass(True)
_sc_pass2 = _sc_pass(False)

TN = 1000


def _tc1_body(p_ref, dg_ref, w_ref, b_ref, h_ref):
    inv = 1.0 / jnp.maximum(dg_ref[0, :, 0] + dg_ref[1, :, 0], 1.0)
    agg = jnp.concatenate([p_ref[0], p_ref[1]], axis=1) * inv[:, None]
    h = jnp.dot(agg, w_ref[...], preferred_element_type=jnp.float32)
    h = jnp.maximum(h + b_ref[...], 0.0)
    h_ref[0] = h[:, :DH]
    h_ref[1] = h[:, DH:]


def _tc1(partials, degp, W1, b1):
    return pl.pallas_call(
        _tc1_body,
        grid=(N // TN,),
        in_specs=[
            pl.BlockSpec((NC, TN, DH), lambda i: (0, i, 0)),
            pl.BlockSpec((NC, TN, 16), lambda i: (0, i, 0)),
            pl.BlockSpec((D, H), lambda i: (0, 0)),
            pl.BlockSpec((1, H), lambda i: (0, 0)),
        ],
        out_specs=pl.BlockSpec((NC, TN, DH), lambda i: (0, i, 0)),
        out_shape=jax.ShapeDtypeStruct((NC, N, DH), jnp.float32),
    )(partials, degp, W1, b1.reshape(1, H))


def _tc2_body(p_ref, dg_ref, w2_ref, b2_ref, wp1_ref, bp1_ref, wp2_ref,
              bp2_ref, loc_ref, g_ref):
    i = pl.program_id(0)
    inv = 1.0 / jnp.maximum(dg_ref[0, :, 0] + dg_ref[1, :, 0], 1.0)
    agg = jnp.concatenate([p_ref[0], p_ref[1]], axis=1) * inv[:, None]
    h2 = jnp.maximum(
        jnp.dot(agg, w2_ref[...], preferred_element_type=jnp.float32) + b2_ref[...], 0.0)
    t = jnp.maximum(
        jnp.dot(h2, wp1_ref[...], preferred_element_type=jnp.float32) + bp1_ref[...], 0.0)
    loc = jnp.dot(t, wp2_ref[...], preferred_element_type=jnp.float32) + bp2_ref[...]
    loc_ref[...] = loc

    @pl.when(i == 0)
    def _():
        g_ref[...] = jnp.zeros_like(g_ref)

    g_ref[...] += jnp.sum(loc, axis=0, keepdims=True) * (1.0 / N)


def _tc2(partials, degp, W2, b2, Wp1, bp1, Wp2, bp2):
    return pl.pallas_call(
        _tc2_body,
        grid=(N // TN,),
        in_specs=[
            pl.BlockSpec((NC, TN, DH), lambda i: (0, i, 0)),
            pl.BlockSpec((NC, TN, 16), lambda i: (0, i, 0)),
            pl.BlockSpec((H, H), lambda i: (0, 0)),
            pl.BlockSpec((1, H), lambda i: (0, 0)),
            pl.BlockSpec((H, PH), lambda i: (0, 0)),
            pl.BlockSpec((1, PH), lambda i: (0, 0)),
            pl.BlockSpec((PH, PO), lambda i: (0, 0)),
            pl.BlockSpec((1, PO), lambda i: (0, 0)),
        ],
        out_specs=[
            pl.BlockSpec((TN, PO), lambda i: (i, 0)),
            pl.BlockSpec((1, PO), lambda i: (0, 0)),
        ],
        out_shape=[
            jax.ShapeDtypeStruct((N, PO), jnp.float32),
            jax.ShapeDtypeStruct((1, PO), jnp.float32),
        ],
    )(partials, degp, W2, b2.reshape(1, H), Wp1, bp1.reshape(1, PH),
      Wp2, bp2.reshape(1, PO))


def kernel(x, edge_index, W1, b1, W2, b2, Wp1, bp1, Wp2, bp2):
    src = edge_index[0]
    dst3 = edge_index[1].reshape(NS, NCHUNK, C)
    xh = jnp.stack([x[:, :DH], x[:, DH:]])
    p1, degp = _sc_pass1(xh, src, dst3)
    h1 = _tc1(p1, degp, W1, b1)
    (p2,) = _sc_pass2(h1, src, dst3)
    loc, ge = _tc2(p2, degp, W2, b2, Wp1, bp1, Wp2, bp2)
    return (ge, loc)


# speedup vs baseline: 9.8799x; 1.0315x over previous
import jax
import jax.numpy as jnp
from jax import lax
from jax.experimental import pallas as pl
from jax.experimental.pallas import tpu as pltpu
from jax.experimental.pallas import tpu_sc as plsc

N = 10000
E = 320000
D = 128
H = 128
PH = 256
PO = 128

NC = 2
NS = 16
DH = D // NC
EW = E // NS
C = 80
NCHUNK = EW // C
NP = 10240
RPS = NP // NS
ZR = 128
B = 5
NG = NCHUNK // B


def _sc_pass(with_deg):
    mesh = plsc.VectorSubcoreMesh(core_axis_name="c", subcore_axis_name="s")
    out_type = [jax.ShapeDtypeStruct((NC, NP, DH), jnp.float32)]
    if with_deg:
        out_type.append(jax.ShapeDtypeStruct((NC, NP, 16), jnp.float32))
    scratch = [
        pltpu.VMEM((EW,), jnp.int32),
        pltpu.VMEM((NCHUNK, C), jnp.int32),
        pltpu.VMEM((B, C, DH), jnp.float32),
        pltpu.VMEM((ZR, DH), jnp.float32),
        pltpu.VMEM((C, 16), jnp.float32),
        pltpu.VMEM((ZR, 16), jnp.float32),
        pltpu.VMEM_SHARED((NP, DH), jnp.float32),
        pltpu.VMEM_SHARED((NP, 16), jnp.float32),
        pltpu.SemaphoreType.DMA,
        pltpu.SemaphoreType.DMA,
        pltpu.SemaphoreType.DMA,
    ]

    def body(xh_hbm, src_hbm, dst_hbm, *rest):
        if with_deg:
            out_acc, out_deg = rest[0], rest[1]
            scr = rest[2:]
        else:
            out_acc = rest[0]
            scr = rest[1:]
        (src_v, dst_v, rows_v, zbuf, ones_v, zdeg, acc_sh, deg_sh,
         gsem, ssem, dsem) = scr

        cid = lax.axis_index("c")
        sid = lax.axis_index("s")

        z16 = jnp.zeros((16,), jnp.float32)
        ones16 = jnp.full((16,), 1.0, jnp.float32)

        pds = [pltpu.async_copy(src_hbm.at[pl.ds(sid * EW, EW)], src_v, gsem),
               pltpu.async_copy(dst_hbm.at[sid], dst_v, gsem)]

        def zero_zbuf(i, carry):
            for j in range(DH // 16):
                zbuf[i, pl.ds(j * 16, 16)] = z16
            return carry

        lax.fori_loop(0, ZR, zero_zbuf, 0)
        if with_deg:
            def fill_ones(i, carry):
                ones_v[i, pl.ds(0, 16)] = ones16
                return carry

            lax.fori_loop(0, C, fill_ones, 0)

            def zero_zdeg(i, carry):
                zdeg[i, pl.ds(0, 16)] = z16
                return carry

            lax.fori_loop(0, ZR, zero_zdeg, 0)
        for k in range(RPS // ZR):
            pds.append(pltpu.async_copy(
                zbuf, acc_sh.at[pl.ds(sid * RPS + k * ZR, ZR)], ssem))
            if with_deg:
                pds.append(pltpu.async_copy(
                    zdeg, deg_sh.at[pl.ds(sid * RPS + k * ZR, ZR)], ssem))
        for d in pds:
            d.wait()
        plsc.subcore_barrier()

        def step(g, carry):
            gds = []
            for b in range(B):
                gds.append(pltpu.async_copy(
                    xh_hbm.at[cid].at[src_v.at[pl.ds((g * B + b) * C, C)]],
                    rows_v.at[b], gsem))
            for b in range(B):
                gds[b].wait()
                if with_deg:
                    @pl.when(cid == g % 2)
                    def _(b=b):
                        pltpu.async_copy(
                            ones_v, deg_sh.at[dst_v.at[g * B + b]], dsem,
                            add=True)
                pltpu.sync_copy(rows_v.at[b],
                                acc_sh.at[dst_v.at[g * B + b]], add=True)
            if with_deg:
                @pl.when(cid == g % 2)
                def _():
                    for b in range(B):
                        pltpu.make_async_copy(
                            ones_v, deg_sh.at[dst_v.at[g * B + b]],
                            dsem).wait()
            return carry

        lax.fori_loop(0, NG, step, 0)
        plsc.subcore_barrier()

        pltpu.sync_copy(acc_sh.at[pl.ds(sid * RPS, RPS)],
                        out_acc.at[cid, pl.ds(sid * RPS, RPS)])
        if with_deg:
            pltpu.sync_copy(deg_sh.at[pl.ds(sid * RPS, RPS)],
                            out_deg.at[cid, pl.ds(sid * RPS, RPS)])

    return pl.kernel(body, out_type=tuple(out_type), mesh=mesh,
                     scratch_types=scratch,
                     compiler_params=pltpu.CompilerParams(
                         use_tc_tiling_on_sc=False))


_sc_pass1 = _sc_pass(True)
_sc_pass2 = _sc_pass(False)

TN = 2000


def _tc1_body(p_ref, dg_ref, w_ref, b_ref, h_ref):
    inv = 1.0 / jnp.maximum(dg_ref[0, :, 0] + dg_ref[1, :, 0], 1.0)
    agg = jnp.concatenate([p_ref[0], p_ref[1]], axis=1) * inv[:, None]
    h = jnp.dot(agg, w_ref[...], preferred_element_type=jnp.float32)
    h = jnp.maximum(h + b_ref[...], 0.0)
    h_ref[0] = h[:, :DH]
    h_ref[1] = h[:, DH:]


def _tc1(partials, degp, W1, b1):
    return pl.pallas_call(
        _tc1_body,
        grid=(N // TN,),
        in_specs=[
            pl.BlockSpec((NC, TN, DH), lambda i: (0, i, 0)),
            pl.BlockSpec((NC, TN, 16), lambda i: (0, i, 0)),
            pl.BlockSpec((D, H), lambda i: (0, 0)),
            pl.BlockSpec((1, H), lambda i: (0, 0)),
        ],
        out_specs=pl.BlockSpec((NC, TN, DH), lambda i: (0, i, 0)),
        out_shape=jax.ShapeDtypeStruct((NC, N, DH), jnp.float32),
    )(partials, degp, W1, b1.reshape(1, H))


def _tc2_body(p_ref, dg_ref, w2_ref, b2_ref, wp1_ref, bp1_ref, wp2_ref,
              bp2_ref, loc_ref, g_ref):
    i = pl.program_id(0)
    inv = 1.0 / jnp.maximum(dg_ref[0, :, 0] + dg_ref[1, :, 0], 1.0)
    agg = jnp.concatenate([p_ref[0], p_ref[1]], axis=1) * inv[:, None]
    h2 = jnp.maximum(
        jnp.dot(agg, w2_ref[...], preferred_element_type=jnp.float32) + b2_ref[...], 0.0)
    t = jnp.maximum(
        jnp.dot(h2, wp1_ref[...], preferred_element_type=jnp.float32) + bp1_ref[...], 0.0)
    loc = jnp.dot(t, wp2_ref[...], preferred_element_type=jnp.float32) + bp2_ref[...]
    loc_ref[...] = loc

    @pl.when(i == 0)
    def _():
        g_ref[...] = jnp.zeros_like(g_ref)

    g_ref[...] += jnp.sum(loc, axis=0, keepdims=True) * (1.0 / N)


def _tc2(partials, degp, W2, b2, Wp1, bp1, Wp2, bp2):
    return pl.pallas_call(
        _tc2_body,
        grid=(N // TN,),
        in_specs=[
            pl.BlockSpec((NC, TN, DH), lambda i: (0, i, 0)),
            pl.BlockSpec((NC, TN, 16), lambda i: (0, i, 0)),
            pl.BlockSpec((H, H), lambda i: (0, 0)),
            pl.BlockSpec((1, H), lambda i: (0, 0)),
            pl.BlockSpec((H, PH), lambda i: (0, 0)),
            pl.BlockSpec((1, PH), lambda i: (0, 0)),
            pl.BlockSpec((PH, PO), lambda i: (0, 0)),
            pl.BlockSpec((1, PO), lambda i: (0, 0)),
        ],
        out_specs=[
            pl.BlockSpec((TN, PO), lambda i: (i, 0)),
            pl.BlockSpec((1, PO), lambda i: (0, 0)),
        ],
        out_shape=[
            jax.ShapeDtypeStruct((N, PO), jnp.float32),
            jax.ShapeDtypeStruct((1, PO), jnp.float32),
        ],
    )(partials, degp, W2, b2.reshape(1, H), Wp1, bp1.reshape(1, PH),
      Wp2, bp2.reshape(1, PO))


def kernel(x, edge_index, W1, b1, W2, b2, Wp1, bp1, Wp2, bp2):
    src = edge_index[0]
    dst3 = edge_index[1].reshape(NS, NCHUNK, C)
    xh = jnp.stack([x[:, :DH], x[:, DH:]])
    p1, degp = _sc_pass1(xh, src, dst3)
    h1 = _tc1(p1, degp, W1, b1)
    (p2,) = _sc_pass2(h1, src, dst3)
    loc, ge = _tc2(p2, degp, W2, b2, Wp1, bp1, Wp2, bp2)
    return (ge, loc)
